# baseline (speedup 1.0000x reference)
import jax
import jax.numpy as jnp
from jax import lax
from jax.experimental import pallas as pl
from jax.experimental.pallas import tpu as pltpu
from jax.experimental.pallas import tpu_sc as plsc

NUM_TOPICS = 512
TOP_K = 8
DIM = 1024

NC = 2
NS = 16
NW = NC * NS
ROWS_PER_W = DIM // NW
SLAB = ROWS_PER_W * NUM_TOPICS


def _tc_topk_body(tvT_ref, out_ref):
    x = tvT_ref[...]
    iota = lax.broadcasted_iota(jnp.int32, x.shape, 0)
    neg_inf = jnp.float32(-jnp.inf)
    for j in range(TOP_K):
        m = jnp.max(x, axis=0, keepdims=True)
        cand = jnp.where(x == m, iota, jnp.int32(DIM))
        amin = jnp.min(cand, axis=0, keepdims=True)
        out_ref[pl.ds(j, 1), :] = amin
        x = jnp.where(cand == amin, neg_inf, x)


def _sc_all_body(idx_hbm, out_hbm, slab, zbuf, idxv, sem, zsem):
    nwords = out_hbm.shape[0]
    batch = nwords // (2 * DIM * NUM_TOPICS)
    half_words = DIM * NUM_TOPICS
    per_w = half_words // 8
    wid = lax.axis_index("s") * NC + lax.axis_index("c")
    b = wid // 8
    rblk = wid % 8
    lo = wid * ROWS_PER_W

    z16 = jnp.zeros((16,), jnp.float32)
    zwords = zbuf.shape[0]
    for i in range(zwords // 16):
        zbuf[pl.ds(i * 16, 16)] = z16
    zbase = b * 2 * half_words + half_words + rblk * per_w
    zcopies = []
    for i in range(per_w // zwords):
        zcopies.append(
            pltpu.async_copy(zbuf, out_hbm.at[pl.ds(zbase + i * zwords, zwords)], zsem)
        )

    for i in range(SLAB // 16):
        slab[pl.ds(i * 16, 16)] = z16
    pltpu.sync_copy(idx_hbm, idxv)
    lane = lax.iota(jnp.int32, 16)
    ones = jnp.ones((16,), jnp.float32)
    for j in range(TOP_K):
        for c in range(NUM_TOPICS // 16):
            idx = idxv[pl.ds(j * NUM_TOPICS + c * 16, 16)]
            t_vec = lane + jnp.int32(c * 16)
            row_local = idx - jnp.int32(lo)
            off = row_local * jnp.int32(NUM_TOPICS) + t_vec
            mask = (idx >= jnp.int32(lo)) & (idx < jnp.int32(lo + ROWS_PER_W))
            plsc.store_scatter(slab, [off], ones, mask=mask)
    copies = []
    for bb in range(batch):
        dst = bb * 2 * half_words + lo * NUM_TOPICS
        copies.append(pltpu.async_copy(slab, out_hbm.at[pl.ds(dst, SLAB)], sem))
    for cp in zcopies + copies:
        cp.wait()


def kernel(inputs, topic_vectors):
    _, batch, max_len, _ = inputs.shape
    tvT = topic_vectors.T

    mesh = plsc.VectorSubcoreMesh(core_axis_name="c", subcore_axis_name="s")

    amins = pl.pallas_call(
        _tc_topk_body,
        in_specs=[pl.BlockSpec(memory_space=pltpu.MemorySpace.VMEM)],
        out_specs=pl.BlockSpec(memory_space=pltpu.MemorySpace.VMEM),
        out_shape=jax.ShapeDtypeStruct((TOP_K, NUM_TOPICS), jnp.int32),
    )(tvT)
    amins_flat = amins.reshape(TOP_K * NUM_TOPICS)

    sc_all = pl.kernel(
        _sc_all_body,
        out_type=jax.ShapeDtypeStruct((batch * max_len * NUM_TOPICS,), jnp.float32),
        mesh=mesh,
        compiler_params=pltpu.CompilerParams(needs_layout_passes=False),
        scratch_types=[
            pltpu.VMEM((SLAB,), jnp.float32),
            pltpu.VMEM((32 * NUM_TOPICS,), jnp.float32),
            pltpu.VMEM((TOP_K * NUM_TOPICS,), jnp.int32),
            pltpu.SemaphoreType.DMA,
            pltpu.SemaphoreType.DMA,
        ],
    )
    out = sc_all(amins_flat)
    return out.reshape(batch, max_len, NUM_TOPICS)
